# Initial kernel scaffold; baseline (speedup 1.0000x reference)
import jax
import jax.numpy as jnp
from jax.experimental import pallas as pl


def kernel(vertices, cat_id, clip_r_feat, clip_t_feat, d0, ste0_w, w1, b1, dir1, ste1_w, c21_w, c21_b, bn1_g, bn1_b, w2, b2, dir2, ste2_w, c22_w, c22_b, bn2_g, bn2_b, w3, b3, dir3, ste3_w, c23_w, c23_b, bn3_g, bn3_b, w4, b4, dir4, ste4_w, c24_w, c24_b):
    raise NotImplementedError("write your pallas kernel here")


# speedup vs baseline: 1.5130x; 1.5130x over previous
import functools

import jax
import jax.numpy as jnp
import numpy as np
from jax import lax
from jax.experimental import pallas as pl
from jax.experimental.pallas import tpu as pltpu
from jax.experimental.pallas import tpu_sc as plsc

_NEI = 10
_S = 7
_OBJ_C = 6
_NW = 32

def _sample(seed, n, k):
    return jax.random.permutation(jax.random.key(seed), n)[:k]


def _knn_body(vq_ref, va_ref, out_ref, *, V):
    vq = vq_ref[0]
    vaT = va_ref[0]
    q2 = jnp.sum(vq * vq, axis=1)[:, None]
    s2 = jnp.sum(vaT * vaT, axis=0)[None, :]
    inner = lax.dot_general(vq.astype(jnp.bfloat16), vaT.astype(jnp.bfloat16),
                            (((1,), (0,)), ((), ())),
                            preferred_element_type=jnp.float32)
    dist = (-2.0 * inner + s2) + q2
    col = lax.broadcasted_iota(jnp.int32, (128, V), 1)
    for it in range(_NEI + 1):
        m = jnp.min(dist, axis=1, keepdims=True)
        cand = jnp.where(dist == m, col, V)
        a = jnp.min(cand, axis=1)
        if it > 0:
            out_ref[0, it - 1, :] = a
        dist = jnp.where(col == a[:, None], jnp.inf, dist)
    zero = jnp.zeros((128,), jnp.int32)
    for it in range(_NEI, 16):
        out_ref[0, it, :] = zero


def _knn10(verts):
    B, V, _ = verts.shape
    Vb = V // 128
    out = pl.pallas_call(
        functools.partial(_knn_body, V=V),
        grid=(B, Vb),
        in_specs=[pl.BlockSpec((1, 128, 3), lambda b, v: (b, v, 0)),
                  pl.BlockSpec((1, 3, V), lambda b, v: (b, 0, 0))],
        out_specs=pl.BlockSpec((1, 16, 128), lambda b, v, _Vb=Vb: (b * _Vb + v, 0, 0)),
        out_shape=jax.ShapeDtypeStruct((B * Vb, 16, 128), jnp.int32),
    )(verts, verts.transpose(0, 2, 1))
    idx = out.reshape(B, Vb, 16, 128).transpose(0, 1, 3, 2).reshape(B, V, 16)
    return idx[:, :, :_NEI]


def _nearest_body(vq_ref, vs_ref, out_ref, *, Vs):
    vq = vq_ref[0]
    vsT = vs_ref[0]
    q2 = jnp.sum(vq * vq, axis=1)[:, None]
    s2 = jnp.sum(vsT * vsT, axis=0)[None, :]
    inner = lax.dot_general(vq.astype(jnp.bfloat16), vsT.astype(jnp.bfloat16),
                            (((1,), (0,)), ((), ())),
                            preferred_element_type=jnp.float32)
    dist = (s2 + q2) - 2.0 * inner
    col = lax.broadcasted_iota(jnp.int32, (128, Vs), 1)
    m = jnp.min(dist, axis=1, keepdims=True)
    a = jnp.min(jnp.where(dist == m, col, Vs), axis=1)
    out_ref[0, 0, :] = a


def _nearest(target, source):
    B, Vt, _ = target.shape
    Vs = source.shape[1]
    Vb = Vt // 128
    out = pl.pallas_call(
        functools.partial(_nearest_body, Vs=Vs),
        grid=(B, Vb),
        in_specs=[pl.BlockSpec((1, 128, 3), lambda b, v: (b, v, 0)),
                  pl.BlockSpec((1, 3, Vs), lambda b, v: (b, 0, 0))],
        out_specs=pl.BlockSpec((1, 1, 128), lambda b, v, _Vb=Vb: (b * _Vb + v, 0, 0)),
        out_shape=jax.ShapeDtypeStruct((B * Vb, 1, 128), jnp.int32),
    )(target, source.transpose(0, 2, 1))
    return out.reshape(B, Vt)


def _surf_body(disp_ref, sup_ref, vq_ref, ste_ref, out_ref):
    disp = disp_ref[0]
    sup = sup_ref[...]
    acc = jnp.full((256, 896), -jnp.inf, jnp.float32)
    for n in range(_NEI):
        dn = disp[:, 3 * n:3 * n + 3]
        th = lax.dot_general(dn, sup, (((1,), (0,)), ((), ())),
                             preferred_element_type=jnp.float32)
        acc = jnp.maximum(acc, jnp.maximum(th, 0.0))
    feat = acc[:, 0:128]
    for s in range(1, _S):
        feat = feat + acc[:, s * 128:(s + 1) * 128]
    f_ste = lax.dot_general(vq_ref[0], ste_ref[...], (((1,), (0,)), ((), ())),
                            preferred_element_type=jnp.float32)
    out_ref[0] = jnp.maximum(feat + f_ste, 0.0)


def _surface(disp_pad, sup, verts, ste0_w):
    B, V, _ = verts.shape
    return pl.pallas_call(
        _surf_body,
        grid=(B, V // 256),
        in_specs=[pl.BlockSpec((1, 256, 32), lambda b, v: (b, v, 0)),
                  pl.BlockSpec((3, _S * 128), lambda b, v: (0, 0)),
                  pl.BlockSpec((1, 256, 3), lambda b, v: (b, v, 0)),
                  pl.BlockSpec((3, 128), lambda b, v: (0, 0))],
        out_specs=pl.BlockSpec((1, 256, 128), lambda b, v: (b, v, 0)),
        out_shape=jax.ShapeDtypeStruct((B, V, 128), jnp.float32),
    )(disp_pad, sup, verts, ste0_w.T)


def _pre_body(x_ref, w_ref, b_ref, ste_ref, fc_ref, fs_ref, fe_ref, *, oc):
    x = x_ref[...]
    fo = jnp.dot(x, w_ref[...], preferred_element_type=jnp.float32) + b_ref[...]
    fc_ref[...] = fo[:, :oc]
    fs_ref[...] = fo[:, oc:]
    fe_ref[...] = jnp.dot(x, ste_ref[...], preferred_element_type=jnp.float32)


def _pre(x, w, b, ste_w, oc):
    P, ic = x.shape
    K = w.shape[1]
    R = min(P, 512)
    return pl.pallas_call(
        functools.partial(_pre_body, oc=oc),
        grid=(P // R,),
        in_specs=[pl.BlockSpec((R, ic), lambda r: (r, 0)),
                  pl.BlockSpec((ic, K), lambda r: (0, 0)),
                  pl.BlockSpec((1, K), lambda r: (0, 0)),
                  pl.BlockSpec((ic, oc), lambda r: (0, 0))],
        out_specs=[pl.BlockSpec((R, oc), lambda r: (r, 0)),
                   pl.BlockSpec((R, K - oc), lambda r: (r, 0)),
                   pl.BlockSpec((R, oc), lambda r: (r, 0))],
        out_shape=[jax.ShapeDtypeStruct((P, oc), jnp.float32),
                   jax.ShapeDtypeStruct((P, K - oc), jnp.float32),
                   jax.ShapeDtypeStruct((P, oc), jnp.float32)],
    )(x, w, b.reshape(1, K), ste_w.T)


def _post_body(act_ref, fc_ref, fe_ref, a_ref, bm_ref, c2b_ref, g_ref, b_ref,
               out_ref, *, B, V, oc, do_bn):
    fuse = fc_ref[...] + act_ref[...]
    f2 = fuse.reshape(B * V, oc)
    gm = jnp.mean(fuse, axis=1)
    y2 = jnp.dot(f2, a_ref[...], preferred_element_type=jnp.float32)
    g2 = jnp.dot(gm, bm_ref[...], preferred_element_type=jnp.float32)
    feat = (y2.reshape(B, V, oc) + g2[:, None, :] + c2b_ref[...][None]
            + fuse + fe_ref[...])
    if do_bn:
        fr = feat.reshape(B * V, oc)
        m = jnp.mean(fr, axis=0)
        v = jnp.mean((fr - m[None, :]) ** 2, axis=0)
        feat = (feat - m[None, None, :]) / jnp.sqrt(v + 1e-5)[None, None, :]
        feat = feat * g_ref[...][None] + b_ref[...][None]
        feat = jnp.maximum(feat, 0.0)
    out_ref[...] = feat


def _post(act, fc, fe, c2_w, c2_b, bn_g, bn_b, do_bn):
    B, V, oc = act.shape
    a = c2_w[:, :oc].T
    bm = c2_w[:, oc:].T
    if bn_g is None:
        bn_g = jnp.ones((oc,), jnp.float32)
        bn_b = jnp.zeros((oc,), jnp.float32)
    return pl.pallas_call(
        functools.partial(_post_body, B=B, V=V, oc=oc, do_bn=do_bn),
        in_specs=[pl.BlockSpec((B, V, oc), lambda: (0, 0, 0)),
                  pl.BlockSpec((B, V, oc), lambda: (0, 0, 0)),
                  pl.BlockSpec((B, V, oc), lambda: (0, 0, 0)),
                  pl.BlockSpec((oc, oc), lambda: (0, 0)),
                  pl.BlockSpec((oc, oc), lambda: (0, 0)),
                  pl.BlockSpec((1, oc), lambda: (0, 0)),
                  pl.BlockSpec((1, oc), lambda: (0, 0)),
                  pl.BlockSpec((1, oc), lambda: (0, 0))],
        out_specs=pl.BlockSpec((B, V, oc), lambda: (0, 0, 0)),
        out_shape=jax.ShapeDtypeStruct((B, V, oc), jnp.float32),
    )(act, fc, fe, a, bm, c2_b.reshape(1, oc), bn_g.reshape(1, oc),
      bn_b.reshape(1, oc))


def _sc_conv(table, idxg, disp_spl, sup_flat, N, W, oc):
    P = idxg.shape[0]
    ppw = P // _NW
    oc16 = oc // 16
    nd = N * 48
    mesh = plsc.VectorSubcoreMesh(core_axis_name="c", subcore_axis_name="s", num_cores=2, num_subcores=16)

    def body(table_h, idx_h, disp_h, sup_h, out_h,
             idx_v, disp_v, sup_v, rows_v, out_v, sem):
        wid = lax.axis_index("s") * 2 + lax.axis_index("c")
        base = wid * ppw
        pltpu.sync_copy(idx_h.at[pl.ds(base, ppw)], idx_v)
        pltpu.sync_copy(disp_h.at[pl.ds(base * nd, ppw * nd)], disp_v)
        pltpu.sync_copy(sup_h, sup_v)

        def point(p, carry):
            idx_row = idx_v[p, :]
            pltpu.async_copy(table_h.at[idx_row], rows_v, sem).wait()
            dbase = p * nd
            spl = [(disp_v[pl.ds(dbase + n * 48, 16)],
                    disp_v[pl.ds(dbase + n * 48 + 16, 16)],
                    disp_v[pl.ds(dbase + n * 48 + 32, 16)]) for n in range(N)]

            def outchunk(c, c2):
                o = jnp.zeros((16,), jnp.float32)
                for s in range(_S):
                    off = s * oc + c * 16
                    s0 = sup_v[pl.ds(off, 16)]
                    s1 = sup_v[pl.ds(W + off, 16)]
                    s2 = sup_v[pl.ds(2 * W + off, 16)]
                    acc = jnp.full((16,), -jnp.inf, jnp.float32)
                    for n in range(N):
                        row = rows_v[n, pl.ds(off, 16)]
                        d0, d1, d2 = spl[n]
                        th = jnp.maximum(d0 * s0 + d1 * s1 + d2 * s2, 0.0)
                        acc = jnp.maximum(acc, th * row)
                    o = o + acc
                out_v[pl.ds(p * oc + c * 16, 16)] = o
                return c2

            lax.fori_loop(0, oc16, outchunk, 0)
            return carry

        lax.fori_loop(0, ppw, point, 0)
        pltpu.sync_copy(out_v, out_h.at[pl.ds(base * oc, ppw * oc)])

    idxg = jnp.pad(idxg, ((0, 0), (0, 16 - N)))
    f = pl.kernel(
        body, out_type=jax.ShapeDtypeStruct((P * oc,), jnp.float32), mesh=mesh,
        scratch_types=[pltpu.VMEM((ppw, 16), jnp.int32),
                       pltpu.VMEM((ppw * nd,), jnp.float32),
                       pltpu.VMEM((3 * W,), jnp.float32),
                       pltpu.VMEM((16, W), jnp.float32),
                       pltpu.VMEM((ppw * oc,), jnp.float32),
                       pltpu.SemaphoreType.DMA])
    return f(table, idxg, disp_spl, sup_flat).reshape(P, oc)


def _sc_pool(table, idxg, N=4):
    P, C = idxg.shape[0], table.shape[1]
    ppw = P // _NW
    C16 = C // 16
    mesh = plsc.VectorSubcoreMesh(core_axis_name="c", subcore_axis_name="s", num_cores=2, num_subcores=16)

    def body(table_h, idx_h, out_h, idx_v, rows_v, out_v, sem):
        wid = lax.axis_index("s") * 2 + lax.axis_index("c")
        base = wid * ppw
        pltpu.sync_copy(idx_h.at[pl.ds(base, ppw)], idx_v)

        def point(p, carry):
            idx_row = idx_v[p, :]
            pltpu.async_copy(table_h.at[idx_row], rows_v, sem).wait()

            def chunk(j, c2):
                acc = rows_v[0, pl.ds(j * 16, 16)]
                for n in range(1, N):
                    acc = jnp.maximum(acc, rows_v[n, pl.ds(j * 16, 16)])
                out_v[pl.ds(p * C + j * 16, 16)] = acc
                return c2

            lax.fori_loop(0, C16, chunk, 0)
            return carry

        lax.fori_loop(0, ppw, point, 0)
        pltpu.sync_copy(out_v, out_h.at[pl.ds(base * C, ppw * C)])

    idxg = jnp.pad(idxg, ((0, 0), (0, 16 - N)))
    f = pl.kernel(
        body, out_type=jax.ShapeDtypeStruct((P * C,), jnp.float32), mesh=mesh,
        scratch_types=[pltpu.VMEM((ppw, 16), jnp.int32),
                       pltpu.VMEM((16, C), jnp.float32),
                       pltpu.VMEM((ppw * C,), jnp.float32),
                       pltpu.SemaphoreType.DMA])
    return f(table, idxg).reshape(P, C)


def _sc_gather(table, idxg):
    P, C = idxg.shape[0], table.shape[1]
    ppw = P // _NW
    mesh = plsc.VectorSubcoreMesh(core_axis_name="c", subcore_axis_name="s", num_cores=2, num_subcores=16)

    def body(table_h, idx_h, out_h, idx_v, rows_v, sem):
        wid = lax.axis_index("s") * 2 + lax.axis_index("c")
        base = wid * ppw
        pltpu.sync_copy(idx_h.at[pl.ds(base, ppw)], idx_v)
        pltpu.async_copy(table_h.at[idx_v], rows_v, sem).wait()
        pltpu.sync_copy(rows_v, out_h.at[pl.ds(base, ppw)])

    f = pl.kernel(
        body, out_type=jax.ShapeDtypeStruct((P, C), jnp.float32), mesh=mesh,
        scratch_types=[pltpu.VMEM((ppw,), jnp.int32),
                       pltpu.VMEM((ppw, C), jnp.float32),
                       pltpu.SemaphoreType.DMA])
    return f(table, idxg)


def _norm_cols(x):
    return x / jnp.maximum(jnp.linalg.norm(x, axis=0, keepdims=True), 1e-12)


def _disp_of(verts, idx):
    nb = jnp.take_along_axis(verts[:, :, None, :], idx[..., None], axis=1)
    d = nb - verts[:, :, None, :]
    return d / jnp.maximum(jnp.linalg.norm(d, axis=-1, keepdims=True), 1e-12)


def _splat(d):
    B, V, N, _ = d.shape
    return jnp.broadcast_to(d[..., None], (B, V, N, 3, 16)).reshape(B * V * N * 48)


def _goffs(idx, V):
    B = idx.shape[0]
    off = (jnp.arange(B, dtype=jnp.int32) * V)[:, None, None]
    return (idx + off).reshape(B * idx.shape[1], -1)


def _layer(verts, idx, fmap, w, b, dirs, ste_w, c2_w, c2_b, bn_g, bn_b, oc, do_bn):
    B, V, ic = fmap.shape
    P = B * V
    fc, fs, fe = _pre(fmap.reshape(P, ic), w, b, ste_w, oc)
    disp = _disp_of(verts, idx)
    act = _sc_conv(fs, _goffs(idx, V), _splat(disp),
                   _norm_cols(dirs).reshape(-1), _NEI, _S * oc, oc)
    return _post(act.reshape(B, V, oc), fc.reshape(B, V, oc),
                 fe.reshape(B, V, oc), c2_w, c2_b, bn_g, bn_b, do_bn)


def kernel(vertices, cat_id, clip_r_feat, clip_t_feat, d0, ste0_w, w1, b1, dir1,
           ste1_w, c21_w, c21_b, bn1_g, bn1_b, w2, b2, dir2, ste2_w, c22_w,
           c22_b, bn2_g, bn2_b, w3, b3, dir3, ste3_w, c23_w, c23_b, bn3_g,
           bn3_b, w4, b4, dir4, ste4_w, c24_w, c24_b):
    B, V, _ = vertices.shape
    idxA = _knn10(vertices)

    dispA = _disp_of(vertices, idxA)
    disp_pad = jnp.pad(dispA.reshape(B, V, _NEI * 3), ((0, 0), (0, 0), (0, 2)))
    fm0 = _surface(disp_pad, _norm_cols(d0), vertices, ste0_w)

    fm1 = _layer(vertices, idxA, fm0, w1, b1, dir1, ste1_w, c21_w, c21_b,
                 bn1_g, bn1_b, 128, True)

    s1 = _sample(100, V, V // 4)
    fmp1 = _sc_pool(fm1.reshape(B * V, 128), _goffs(idxA[:, s1, :4], V))
    vp1 = vertices[:, s1, :]
    V1 = V // 4

    idxB = _knn10(vp1)
    fm2 = _layer(vp1, idxB, fmp1.reshape(B, V1, 128), w2, b2, dir2, ste2_w,
                 c22_w, c22_b, bn2_g, bn2_b, 256, True)
    fm3 = _layer(vp1, idxB, fm2, w3, b3, dir3, ste3_w, c23_w, c23_b,
                 bn3_g, bn3_b, 256, True)

    s2 = _sample(101, V1, V1 // 4)
    fmp2 = _sc_pool(fm3.reshape(B * V1, 256), _goffs(idxB[:, s2, :4], V1))
    vp2 = vp1[:, s2, :]
    V2 = V1 // 4

    idxC = _knn10(vp2)
    fm4 = _layer(vp2, idxC, fmp2.reshape(B, V2, 256), w4, b4, dir4, ste4_w,
                 c24_w, c24_b, None, None, 512, False)

    np1 = _nearest(vertices, vp1)
    np2 = _nearest(vertices, vp2)
    t23 = jnp.concatenate([fm2, fm3], axis=-1).reshape(B * V1, 512)
    fm23u = _sc_gather(t23, _goffs(np1[..., None], V1)[:, 0])
    fm4u = _sc_gather(fm4.reshape(B * V2, 512), _goffs(np2[..., None], V2)[:, 0])

    oh = (cat_id == jnp.arange(_OBJ_C, dtype=cat_id.dtype)[None, :]).astype(jnp.float32)
    oh = jnp.broadcast_to(oh[:, None, :], (B, V, _OBJ_C))
    return jnp.concatenate([fm0, fm1, fm23u.reshape(B, V, 512),
                            fm4u.reshape(B, V, 512), oh], axis=2)


# speedup vs baseline: 1.5221x; 1.0060x over previous
import functools

import jax
import jax.numpy as jnp
import numpy as np
from jax import lax
from jax.experimental import pallas as pl
from jax.experimental.pallas import tpu as pltpu
from jax.experimental.pallas import tpu_sc as plsc

_NEI = 10
_S = 7
_OBJ_C = 6
_NW = 32

def _sample(seed, n, k):
    return jax.random.permutation(jax.random.key(seed), n)[:k]


def _knn_body(vq_ref, va_ref, out_ref, *, V):
    vq = vq_ref[0]
    vaT = va_ref[0]
    q2 = jnp.sum(vq * vq, axis=1)[:, None]
    s2 = jnp.sum(vaT * vaT, axis=0)[None, :]
    inner = lax.dot_general(vq.astype(jnp.bfloat16), vaT.astype(jnp.bfloat16),
                            (((1,), (0,)), ((), ())),
                            preferred_element_type=jnp.float32)
    dist = (-2.0 * inner + s2) + q2
    col = lax.broadcasted_iota(jnp.int32, (128, V), 1)
    for it in range(_NEI + 1):
        m = jnp.min(dist, axis=1, keepdims=True)
        cand = jnp.where(dist == m, col, V)
        a = jnp.min(cand, axis=1)
        if it > 0:
            out_ref[0, it - 1, :] = a
        dist = jnp.where(col == a[:, None], jnp.inf, dist)
    zero = jnp.zeros((128,), jnp.int32)
    for it in range(_NEI, 16):
        out_ref[0, it, :] = zero


def _knn10(verts):
    B, V, _ = verts.shape
    Vb = V // 128
    out = pl.pallas_call(
        functools.partial(_knn_body, V=V),
        grid=(B, Vb),
        in_specs=[pl.BlockSpec((1, 128, 3), lambda b, v: (b, v, 0)),
                  pl.BlockSpec((1, 3, V), lambda b, v: (b, 0, 0))],
        out_specs=pl.BlockSpec((1, 16, 128), lambda b, v, _Vb=Vb: (b * _Vb + v, 0, 0)),
        out_shape=jax.ShapeDtypeStruct((B * Vb, 16, 128), jnp.int32),
    )(verts, verts.transpose(0, 2, 1))
    idx = out.reshape(B, Vb, 16, 128).transpose(0, 1, 3, 2).reshape(B, V, 16)
    return idx[:, :, :_NEI]


def _nearest_body(vq_ref, vs_ref, out_ref, *, Vs):
    vq = vq_ref[0]
    vsT = vs_ref[0]
    q2 = jnp.sum(vq * vq, axis=1)[:, None]
    s2 = jnp.sum(vsT * vsT, axis=0)[None, :]
    inner = lax.dot_general(vq.astype(jnp.bfloat16), vsT.astype(jnp.bfloat16),
                            (((1,), (0,)), ((), ())),
                            preferred_element_type=jnp.float32)
    dist = (s2 + q2) - 2.0 * inner
    col = lax.broadcasted_iota(jnp.int32, (128, Vs), 1)
    m = jnp.min(dist, axis=1, keepdims=True)
    a = jnp.min(jnp.where(dist == m, col, Vs), axis=1)
    out_ref[0, 0, :] = a


def _nearest(target, source):
    B, Vt, _ = target.shape
    Vs = source.shape[1]
    Vb = Vt // 128
    out = pl.pallas_call(
        functools.partial(_nearest_body, Vs=Vs),
        grid=(B, Vb),
        in_specs=[pl.BlockSpec((1, 128, 3), lambda b, v: (b, v, 0)),
                  pl.BlockSpec((1, 3, Vs), lambda b, v: (b, 0, 0))],
        out_specs=pl.BlockSpec((1, 1, 128), lambda b, v, _Vb=Vb: (b * _Vb + v, 0, 0)),
        out_shape=jax.ShapeDtypeStruct((B * Vb, 1, 128), jnp.int32),
    )(target, source.transpose(0, 2, 1))
    return out.reshape(B, Vt)


def _surf_body(disp_ref, sup_ref, vq_ref, ste_ref, out_ref):
    disp = disp_ref[0]
    sup = sup_ref[...]
    acc = jnp.full((256, 896), -jnp.inf, jnp.float32)
    for n in range(_NEI):
        dn = disp[:, 3 * n:3 * n + 3]
        th = lax.dot_general(dn, sup, (((1,), (0,)), ((), ())),
                             preferred_element_type=jnp.float32)
        acc = jnp.maximum(acc, jnp.maximum(th, 0.0))
    feat = acc[:, 0:128]
    for s in range(1, _S):
        feat = feat + acc[:, s * 128:(s + 1) * 128]
    f_ste = lax.dot_general(vq_ref[0], ste_ref[...], (((1,), (0,)), ((), ())),
                            preferred_element_type=jnp.float32)
    out_ref[0] = jnp.maximum(feat + f_ste, 0.0)


def _surface(disp_pad, sup, verts, ste0_w):
    B, V, _ = verts.shape
    return pl.pallas_call(
        _surf_body,
        grid=(B, V // 256),
        in_specs=[pl.BlockSpec((1, 256, 32), lambda b, v: (b, v, 0)),
                  pl.BlockSpec((3, _S * 128), lambda b, v: (0, 0)),
                  pl.BlockSpec((1, 256, 3), lambda b, v: (b, v, 0)),
                  pl.BlockSpec((3, 128), lambda b, v: (0, 0))],
        out_specs=pl.BlockSpec((1, 256, 128), lambda b, v: (b, v, 0)),
        out_shape=jax.ShapeDtypeStruct((B, V, 128), jnp.float32),
    )(disp_pad, sup, verts, ste0_w.T)


def _pre_body(x_ref, w_ref, b_ref, ste_ref, fc_ref, fs_ref, fe_ref, *, oc):
    x = x_ref[...]
    fo = jnp.dot(x, w_ref[...], preferred_element_type=jnp.float32) + b_ref[...]
    fc_ref[...] = fo[:, :oc]
    fs_ref[...] = fo[:, oc:]
    fe_ref[...] = jnp.dot(x, ste_ref[...], preferred_element_type=jnp.float32)


def _pre(x, w, b, ste_w, oc):
    P, ic = x.shape
    K = w.shape[1]
    R = min(P, 512)
    return pl.pallas_call(
        functools.partial(_pre_body, oc=oc),
        grid=(P // R,),
        in_specs=[pl.BlockSpec((R, ic), lambda r: (r, 0)),
                  pl.BlockSpec((ic, K), lambda r: (0, 0)),
                  pl.BlockSpec((1, K), lambda r: (0, 0)),
                  pl.BlockSpec((ic, oc), lambda r: (0, 0))],
        out_specs=[pl.BlockSpec((R, oc), lambda r: (r, 0)),
                   pl.BlockSpec((R, K - oc), lambda r: (r, 0)),
                   pl.BlockSpec((R, oc), lambda r: (r, 0))],
        out_shape=[jax.ShapeDtypeStruct((P, oc), jnp.float32),
                   jax.ShapeDtypeStruct((P, K - oc), jnp.float32),
                   jax.ShapeDtypeStruct((P, oc), jnp.float32)],
    )(x, w, b.reshape(1, K), ste_w.T)


def _post_body(act_ref, fc_ref, fe_ref, a_ref, bm_ref, c2b_ref, g_ref, b_ref,
               out_ref, *, B, V, oc, do_bn):
    fuse = fc_ref[...] + act_ref[...]
    f2 = fuse.reshape(B * V, oc)
    gm = jnp.mean(fuse, axis=1)
    y2 = jnp.dot(f2, a_ref[...], preferred_element_type=jnp.float32)
    g2 = jnp.dot(gm, bm_ref[...], preferred_element_type=jnp.float32)
    feat = (y2.reshape(B, V, oc) + g2[:, None, :] + c2b_ref[...][None]
            + fuse + fe_ref[...])
    if do_bn:
        fr = feat.reshape(B * V, oc)
        m = jnp.mean(fr, axis=0)
        v = jnp.mean((fr - m[None, :]) ** 2, axis=0)
        feat = (feat - m[None, None, :]) / jnp.sqrt(v + 1e-5)[None, None, :]
        feat = feat * g_ref[...][None] + b_ref[...][None]
        feat = jnp.maximum(feat, 0.0)
    out_ref[...] = feat


def _post(act, fc, fe, c2_w, c2_b, bn_g, bn_b, do_bn):
    B, V, oc = act.shape
    a = c2_w[:, :oc].T
    bm = c2_w[:, oc:].T
    if bn_g is None:
        bn_g = jnp.ones((oc,), jnp.float32)
        bn_b = jnp.zeros((oc,), jnp.float32)
    return pl.pallas_call(
        functools.partial(_post_body, B=B, V=V, oc=oc, do_bn=do_bn),
        in_specs=[pl.BlockSpec((B, V, oc), lambda: (0, 0, 0)),
                  pl.BlockSpec((B, V, oc), lambda: (0, 0, 0)),
                  pl.BlockSpec((B, V, oc), lambda: (0, 0, 0)),
                  pl.BlockSpec((oc, oc), lambda: (0, 0)),
                  pl.BlockSpec((oc, oc), lambda: (0, 0)),
                  pl.BlockSpec((1, oc), lambda: (0, 0)),
                  pl.BlockSpec((1, oc), lambda: (0, 0)),
                  pl.BlockSpec((1, oc), lambda: (0, 0))],
        out_specs=pl.BlockSpec((B, V, oc), lambda: (0, 0, 0)),
        out_shape=jax.ShapeDtypeStruct((B, V, oc), jnp.float32),
    )(act, fc, fe, a, bm, c2_b.reshape(1, oc), bn_g.reshape(1, oc),
      bn_b.reshape(1, oc))


def _sc_conv(table, idxg, disp_spl, sup_flat, N, W, oc):
    P = idxg.shape[0]
    ppw = P // _NW
    oc16 = oc // 16
    nd = N * 48
    db = (2 * 16 * W + ppw * nd + 3 * W + ppw * oc + ppw * 16) * 4 < 500_000
    mesh = plsc.VectorSubcoreMesh(core_axis_name="c", subcore_axis_name="s", num_cores=2, num_subcores=16)

    def body(table_h, idx_h, disp_h, sup_h, out_h,
             idx_v, disp_v, sup_v, rows0_v, rows1_v, out_v, sem0, sem1):
        wid = lax.axis_index("s") * 2 + lax.axis_index("c")
        base = wid * ppw
        pltpu.sync_copy(idx_h.at[pl.ds(base, ppw)], idx_v)
        pltpu.sync_copy(disp_h.at[pl.ds(base * nd, ppw * nd)], disp_v)
        pltpu.sync_copy(sup_h, sup_v)

        def start(p, buf, sem):
            pc = jnp.minimum(p, ppw - 1)
            pltpu.make_async_copy(table_h.at[idx_v[pc, :]], buf, sem).start()

        def compute(p, buf, sem):
            pltpu.make_async_copy(table_h.at[idx_v[0, :]], buf, sem).wait()
            dbase = p * nd
            spl = [(disp_v[pl.ds(dbase + n * 48, 16)],
                    disp_v[pl.ds(dbase + n * 48 + 16, 16)],
                    disp_v[pl.ds(dbase + n * 48 + 32, 16)]) for n in range(N)]

            def outchunk(c, c2):
                o = jnp.zeros((16,), jnp.float32)
                for s in range(_S):
                    off = s * oc + c * 16
                    s0 = sup_v[pl.ds(off, 16)]
                    s1 = sup_v[pl.ds(W + off, 16)]
                    s2 = sup_v[pl.ds(2 * W + off, 16)]
                    acc = jnp.full((16,), -jnp.inf, jnp.float32)
                    for n in range(N):
                        row = buf[n, pl.ds(off, 16)]
                        d0, d1, d2 = spl[n]
                        th = jnp.maximum(d0 * s0 + d1 * s1 + d2 * s2, 0.0)
                        acc = jnp.maximum(acc, th * row)
                    o = o + acc
                out_v[pl.ds(p * oc + c * 16, 16)] = o
                return c2

            lax.fori_loop(0, oc16, outchunk, 0)

        start(0, rows0_v, sem0)
        if db:
            def pair(p2, carry):
                p = p2 * 2
                start(p + 1, rows1_v, sem1)
                compute(p, rows0_v, sem0)
                start(p + 2, rows0_v, sem0)
                compute(p + 1, rows1_v, sem1)
                return carry

            lax.fori_loop(0, ppw // 2, pair, 0)
            pltpu.make_async_copy(table_h.at[idx_v[0, :]], rows0_v, sem0).wait()
        else:
            def point(p, carry):
                compute(p, rows0_v, sem0)
                start(p + 1, rows0_v, sem0)
                return carry

            lax.fori_loop(0, ppw, point, 0)
            pltpu.make_async_copy(table_h.at[idx_v[0, :]], rows0_v, sem0).wait()
        pltpu.sync_copy(out_v, out_h.at[pl.ds(base * oc, ppw * oc)])

    idxg = jnp.pad(idxg, ((0, 0), (0, 16 - N)))
    rows1_shape = (16, W) if db else (1, 16)
    f = pl.kernel(
        body, out_type=jax.ShapeDtypeStruct((P * oc,), jnp.float32), mesh=mesh,
        scratch_types=[pltpu.VMEM((ppw, 16), jnp.int32),
                       pltpu.VMEM((ppw * nd,), jnp.float32),
                       pltpu.VMEM((3 * W,), jnp.float32),
                       pltpu.VMEM((16, W), jnp.float32),
                       pltpu.VMEM(rows1_shape, jnp.float32),
                       pltpu.VMEM((ppw * oc,), jnp.float32),
                       pltpu.SemaphoreType.DMA,
                       pltpu.SemaphoreType.DMA])
    return f(table, idxg, disp_spl, sup_flat).reshape(P, oc)


def _sc_pool(table, idxg, N=4):
    P, C = idxg.shape[0], table.shape[1]
    ppw = P // _NW
    C16 = C // 16
    mesh = plsc.VectorSubcoreMesh(core_axis_name="c", subcore_axis_name="s", num_cores=2, num_subcores=16)

    def body(table_h, idx_h, out_h, idx_v, rows_v, out_v, sem):
        wid = lax.axis_index("s") * 2 + lax.axis_index("c")
        base = wid * ppw
        pltpu.sync_copy(idx_h.at[pl.ds(base, ppw)], idx_v)

        def point(p, carry):
            idx_row = idx_v[p, :]
            pltpu.async_copy(table_h.at[idx_row], rows_v, sem).wait()

            def chunk(j, c2):
                acc = rows_v[0, pl.ds(j * 16, 16)]
                for n in range(1, N):
                    acc = jnp.maximum(acc, rows_v[n, pl.ds(j * 16, 16)])
                out_v[pl.ds(p * C + j * 16, 16)] = acc
                return c2

            lax.fori_loop(0, C16, chunk, 0)
            return carry

        lax.fori_loop(0, ppw, point, 0)
        pltpu.sync_copy(out_v, out_h.at[pl.ds(base * C, ppw * C)])

    idxg = jnp.pad(idxg, ((0, 0), (0, 16 - N)))
    f = pl.kernel(
        body, out_type=jax.ShapeDtypeStruct((P * C,), jnp.float32), mesh=mesh,
        scratch_types=[pltpu.VMEM((ppw, 16), jnp.int32),
                       pltpu.VMEM((16, C), jnp.float32),
                       pltpu.VMEM((ppw * C,), jnp.float32),
                       pltpu.SemaphoreType.DMA])
    return f(table, idxg).reshape(P, C)


def _sc_gather(table, idxg):
    P, C = idxg.shape[0], table.shape[1]
    ppw = P // _NW
    mesh = plsc.VectorSubcoreMesh(core_axis_name="c", subcore_axis_name="s", num_cores=2, num_subcores=16)

    def body(table_h, idx_h, out_h, idx_v, rows_v, sem):
        wid = lax.axis_index("s") * 2 + lax.axis_index("c")
        base = wid * ppw
        pltpu.sync_copy(idx_h.at[pl.ds(base, ppw)], idx_v)
        pltpu.async_copy(table_h.at[idx_v], rows_v, sem).wait()
        pltpu.sync_copy(rows_v, out_h.at[pl.ds(base, ppw)])

    f = pl.kernel(
        body, out_type=jax.ShapeDtypeStruct((P, C), jnp.float32), mesh=mesh,
        scratch_types=[pltpu.VMEM((ppw,), jnp.int32),
                       pltpu.VMEM((ppw, C), jnp.float32),
                       pltpu.SemaphoreType.DMA])
    return f(table, idxg)


def _norm_cols(x):
    return x / jnp.maximum(jnp.linalg.norm(x, axis=0, keepdims=True), 1e-12)


def _disp_of(verts, idx):
    nb = jnp.take_along_axis(verts[:, :, None, :], idx[..., None], axis=1)
    d = nb - verts[:, :, None, :]
    return d / jnp.maximum(jnp.linalg.norm(d, axis=-1, keepdims=True), 1e-12)


def _splat(d):
    B, V, N, _ = d.shape
    return jnp.broadcast_to(d[..., None], (B, V, N, 3, 16)).reshape(B * V * N * 48)


def _goffs(idx, V):
    B = idx.shape[0]
    off = (jnp.arange(B, dtype=jnp.int32) * V)[:, None, None]
    return (idx + off).reshape(B * idx.shape[1], -1)


def _layer(verts, idx, fmap, w, b, dirs, ste_w, c2_w, c2_b, bn_g, bn_b, oc, do_bn):
    B, V, ic = fmap.shape
    P = B * V
    fc, fs, fe = _pre(fmap.reshape(P, ic), w, b, ste_w, oc)
    disp = _disp_of(verts, idx)
    act = _sc_conv(fs, _goffs(idx, V), _splat(disp),
                   _norm_cols(dirs).reshape(-1), _NEI, _S * oc, oc)
    return _post(act.reshape(B, V, oc), fc.reshape(B, V, oc),
                 fe.reshape(B, V, oc), c2_w, c2_b, bn_g, bn_b, do_bn)


def kernel(vertices, cat_id, clip_r_feat, clip_t_feat, d0, ste0_w, w1, b1, dir1,
           ste1_w, c21_w, c21_b, bn1_g, bn1_b, w2, b2, dir2, ste2_w, c22_w,
           c22_b, bn2_g, bn2_b, w3, b3, dir3, ste3_w, c23_w, c23_b, bn3_g,
           bn3_b, w4, b4, dir4, ste4_w, c24_w, c24_b):
    B, V, _ = vertices.shape
    idxA = _knn10(vertices)

    dispA = _disp_of(vertices, idxA)
    disp_pad = jnp.pad(dispA.reshape(B, V, _NEI * 3), ((0, 0), (0, 0), (0, 2)))
    fm0 = _surface(disp_pad, _norm_cols(d0), vertices, ste0_w)

    fm1 = _layer(vertices, idxA, fm0, w1, b1, dir1, ste1_w, c21_w, c21_b,
                 bn1_g, bn1_b, 128, True)

    s1 = _sample(100, V, V // 4)
    fmp1 = _sc_pool(fm1.reshape(B * V, 128), _goffs(idxA[:, s1, :4], V))
    vp1 = vertices[:, s1, :]
    V1 = V // 4

    idxB = _knn10(vp1)
    fm2 = _layer(vp1, idxB, fmp1.reshape(B, V1, 128), w2, b2, dir2, ste2_w,
                 c22_w, c22_b, bn2_g, bn2_b, 256, True)
    fm3 = _layer(vp1, idxB, fm2, w3, b3, dir3, ste3_w, c23_w, c23_b,
                 bn3_g, bn3_b, 256, True)

    s2 = _sample(101, V1, V1 // 4)
    fmp2 = _sc_pool(fm3.reshape(B * V1, 256), _goffs(idxB[:, s2, :4], V1))
    vp2 = vp1[:, s2, :]
    V2 = V1 // 4

    idxC = _knn10(vp2)
    fm4 = _layer(vp2, idxC, fmp2.reshape(B, V2, 256), w4, b4, dir4, ste4_w,
                 c24_w, c24_b, None, None, 512, False)

    np1 = _nearest(vertices, vp1)
    np2 = _nearest(vertices, vp2)
    t23 = jnp.concatenate([fm2, fm3], axis=-1).reshape(B * V1, 512)
    fm23u = _sc_gather(t23, _goffs(np1[..., None], V1)[:, 0])
    fm4u = _sc_gather(fm4.reshape(B * V2, 512), _goffs(np2[..., None], V2)[:, 0])

    oh = (cat_id == jnp.arange(_OBJ_C, dtype=cat_id.dtype)[None, :]).astype(jnp.float32)
    oh = jnp.broadcast_to(oh[:, None, :], (B, V, _OBJ_C))
    return jnp.concatenate([fm0, fm1, fm23u.reshape(B, V, 512),
                            fm4u.reshape(B, V, 512), oh], axis=2)


# speedup vs baseline: 1.5387x; 1.0109x over previous
import functools

import jax
import jax.numpy as jnp
import numpy as np
from jax import lax
from jax.experimental import pallas as pl
from jax.experimental.pallas import tpu as pltpu
from jax.experimental.pallas import tpu_sc as plsc

_NEI = 10
_S = 7
_OBJ_C = 6
_NW = 32

def _sample(seed, n, k):
    return jax.random.permutation(jax.random.key(seed), n)[:k]


def _knn_body(vq_ref, va_ref, out_ref, *, V):
    vq = vq_ref[0]
    vaT = va_ref[0]
    q2 = jnp.sum(vq * vq, axis=1)[:, None]
    s2 = jnp.sum(vaT * vaT, axis=0)[None, :]
    inner = lax.dot_general(vq.astype(jnp.bfloat16), vaT.astype(jnp.bfloat16),
                            (((1,), (0,)), ((), ())),
                            preferred_element_type=jnp.float32)
    dist = (-2.0 * inner + s2) + q2
    col = lax.broadcasted_iota(jnp.int32, (128, V), 1)
    for it in range(_NEI + 1):
        m = jnp.min(dist, axis=1, keepdims=True)
        cand = jnp.where(dist == m, col, V)
        a = jnp.min(cand, axis=1)
        if it > 0:
            out_ref[0, it - 1, :] = a
        dist = jnp.where(col == a[:, None], jnp.inf, dist)
    zero = jnp.zeros((128,), jnp.int32)
    for it in range(_NEI, 16):
        out_ref[0, it, :] = zero


def _knn10(verts):
    B, V, _ = verts.shape
    Vb = V // 128
    out = pl.pallas_call(
        functools.partial(_knn_body, V=V),
        grid=(B, Vb),
        in_specs=[pl.BlockSpec((1, 128, 3), lambda b, v: (b, v, 0)),
                  pl.BlockSpec((1, 3, V), lambda b, v: (b, 0, 0))],
        out_specs=pl.BlockSpec((1, 16, 128), lambda b, v, _Vb=Vb: (b * _Vb + v, 0, 0)),
        out_shape=jax.ShapeDtypeStruct((B * Vb, 16, 128), jnp.int32),
    )(verts, verts.transpose(0, 2, 1))
    idx = out.reshape(B, Vb, 16, 128).transpose(0, 1, 3, 2).reshape(B, V, 16)
    return idx[:, :, :_NEI]


def _nearest_body(vq_ref, vs_ref, out_ref, *, Vs):
    vq = vq_ref[0]
    vsT = vs_ref[0]
    q2 = jnp.sum(vq * vq, axis=1)[:, None]
    s2 = jnp.sum(vsT * vsT, axis=0)[None, :]
    inner = lax.dot_general(vq.astype(jnp.bfloat16), vsT.astype(jnp.bfloat16),
                            (((1,), (0,)), ((), ())),
                            preferred_element_type=jnp.float32)
    dist = (s2 + q2) - 2.0 * inner
    col = lax.broadcasted_iota(jnp.int32, (128, Vs), 1)
    m = jnp.min(dist, axis=1, keepdims=True)
    a = jnp.min(jnp.where(dist == m, col, Vs), axis=1)
    out_ref[0, 0, :] = a


def _nearest(target, source):
    B, Vt, _ = target.shape
    Vs = source.shape[1]
    Vb = Vt // 128
    out = pl.pallas_call(
        functools.partial(_nearest_body, Vs=Vs),
        grid=(B, Vb),
        in_specs=[pl.BlockSpec((1, 128, 3), lambda b, v: (b, v, 0)),
                  pl.BlockSpec((1, 3, Vs), lambda b, v: (b, 0, 0))],
        out_specs=pl.BlockSpec((1, 1, 128), lambda b, v, _Vb=Vb: (b * _Vb + v, 0, 0)),
        out_shape=jax.ShapeDtypeStruct((B * Vb, 1, 128), jnp.int32),
    )(target, source.transpose(0, 2, 1))
    return out.reshape(B, Vt)


def _surf_body(disp_ref, sup_ref, vq_ref, ste_ref, out_ref):
    disp = disp_ref[0]
    sup = sup_ref[...]
    acc = jnp.full((256, 896), -jnp.inf, jnp.float32)
    for n in range(_NEI):
        dn = disp[:, 3 * n:3 * n + 3]
        th = lax.dot_general(dn, sup, (((1,), (0,)), ((), ())),
                             preferred_element_type=jnp.float32)
        acc = jnp.maximum(acc, jnp.maximum(th, 0.0))
    feat = acc[:, 0:128]
    for s in range(1, _S):
        feat = feat + acc[:, s * 128:(s + 1) * 128]
    f_ste = lax.dot_general(vq_ref[0], ste_ref[...], (((1,), (0,)), ((), ())),
                            preferred_element_type=jnp.float32)
    out_ref[0] = jnp.maximum(feat + f_ste, 0.0)


def _surface(disp_pad, sup, verts, ste0_w):
    B, V, _ = verts.shape
    return pl.pallas_call(
        _surf_body,
        grid=(B, V // 256),
        in_specs=[pl.BlockSpec((1, 256, 32), lambda b, v: (b, v, 0)),
                  pl.BlockSpec((3, _S * 128), lambda b, v: (0, 0)),
                  pl.BlockSpec((1, 256, 3), lambda b, v: (b, v, 0)),
                  pl.BlockSpec((3, 128), lambda b, v: (0, 0))],
        out_specs=pl.BlockSpec((1, 256, 128), lambda b, v: (b, v, 0)),
        out_shape=jax.ShapeDtypeStruct((B, V, 128), jnp.float32),
    )(disp_pad, sup, verts, ste0_w.T)


def _pre_body(x_ref, w_ref, b_ref, ste_ref, fc_ref, fs_ref, fe_ref, *, oc):
    x = x_ref[...]
    fo = jnp.dot(x, w_ref[...], preferred_element_type=jnp.float32) + b_ref[...]
    fc_ref[...] = fo[:, :oc]
    fs_ref[...] = fo[:, oc:]
    fe_ref[...] = jnp.dot(x, ste_ref[...], preferred_element_type=jnp.float32)


def _pre(x, w, b, ste_w, oc):
    P, ic = x.shape
    K = w.shape[1]
    R = min(P, 512)
    return pl.pallas_call(
        functools.partial(_pre_body, oc=oc),
        grid=(P // R,),
        in_specs=[pl.BlockSpec((R, ic), lambda r: (r, 0)),
                  pl.BlockSpec((ic, K), lambda r: (0, 0)),
                  pl.BlockSpec((1, K), lambda r: (0, 0)),
                  pl.BlockSpec((ic, oc), lambda r: (0, 0))],
        out_specs=[pl.BlockSpec((R, oc), lambda r: (r, 0)),
                   pl.BlockSpec((R, K - oc), lambda r: (r, 0)),
                   pl.BlockSpec((R, oc), lambda r: (r, 0))],
        out_shape=[jax.ShapeDtypeStruct((P, oc), jnp.float32),
                   jax.ShapeDtypeStruct((P, K - oc), jnp.float32),
                   jax.ShapeDtypeStruct((P, oc), jnp.float32)],
    )(x, w, b.reshape(1, K), ste_w.T)


def _post_body(act_ref, fc_ref, fe_ref, a_ref, bm_ref, c2b_ref, g_ref, b_ref,
               out_ref, *, B, V, oc, do_bn):
    fuse = fc_ref[...] + act_ref[...]
    f2 = fuse.reshape(B * V, oc)
    gm = jnp.mean(fuse, axis=1)
    y2 = jnp.dot(f2, a_ref[...], preferred_element_type=jnp.float32)
    g2 = jnp.dot(gm, bm_ref[...], preferred_element_type=jnp.float32)
    feat = (y2.reshape(B, V, oc) + g2[:, None, :] + c2b_ref[...][None]
            + fuse + fe_ref[...])
    if do_bn:
        fr = feat.reshape(B * V, oc)
        m = jnp.mean(fr, axis=0)
        v = jnp.mean((fr - m[None, :]) ** 2, axis=0)
        feat = (feat - m[None, None, :]) / jnp.sqrt(v + 1e-5)[None, None, :]
        feat = feat * g_ref[...][None] + b_ref[...][None]
        feat = jnp.maximum(feat, 0.0)
    out_ref[...] = feat


def _post(act, fc, fe, c2_w, c2_b, bn_g, bn_b, do_bn):
    B, V, oc = act.shape
    a = c2_w[:, :oc].T
    bm = c2_w[:, oc:].T
    if bn_g is None:
        bn_g = jnp.ones((oc,), jnp.float32)
        bn_b = jnp.zeros((oc,), jnp.float32)
    return pl.pallas_call(
        functools.partial(_post_body, B=B, V=V, oc=oc, do_bn=do_bn),
        in_specs=[pl.BlockSpec((B, V, oc), lambda: (0, 0, 0)),
                  pl.BlockSpec((B, V, oc), lambda: (0, 0, 0)),
                  pl.BlockSpec((B, V, oc), lambda: (0, 0, 0)),
                  pl.BlockSpec((oc, oc), lambda: (0, 0)),
                  pl.BlockSpec((oc, oc), lambda: (0, 0)),
                  pl.BlockSpec((1, oc), lambda: (0, 0)),
                  pl.BlockSpec((1, oc), lambda: (0, 0)),
                  pl.BlockSpec((1, oc), lambda: (0, 0))],
        out_specs=pl.BlockSpec((B, V, oc), lambda: (0, 0, 0)),
        out_shape=jax.ShapeDtypeStruct((B, V, oc), jnp.float32),
    )(act, fc, fe, a, bm, c2_b.reshape(1, oc), bn_g.reshape(1, oc),
      bn_b.reshape(1, oc))


def _sc_conv(table, idxg, disp_spl, sup_flat, N, W, oc):
    P = idxg.shape[0]
    ppw = P // _NW
    oc16 = oc // 16
    nd = N * 48
    db = (2 * 16 * W + ppw * nd + 3 * W + ppw * oc + ppw * 16) * 4 < 500_000
    mesh = plsc.VectorSubcoreMesh(core_axis_name="c", subcore_axis_name="s", num_cores=2, num_subcores=16)

    def body(table_h, idx_h, disp_h, sup_h, out_h,
             idx_v, disp_v, sup_v, rows0_v, rows1_v, out_v, sem0, sem1):
        wid = lax.axis_index("s") * 2 + lax.axis_index("c")
        base = wid * ppw
        pltpu.sync_copy(idx_h.at[pl.ds(base, ppw)], idx_v)
        pltpu.sync_copy(disp_h.at[pl.ds(base * nd, ppw * nd)], disp_v)
        pltpu.sync_copy(sup_h, sup_v)

        def start(p, buf, sem):
            pc = jnp.minimum(p, ppw - 1)
            pltpu.make_async_copy(table_h.at[idx_v[pc, :]], buf, sem).start()

        def compute(p, buf, sem):
            pltpu.make_async_copy(table_h.at[idx_v[0, :]], buf, sem).wait()
            dbase = p * nd
            spl = [(disp_v[pl.ds(dbase + n * 48, 16)],
                    disp_v[pl.ds(dbase + n * 48 + 16, 16)],
                    disp_v[pl.ds(dbase + n * 48 + 32, 16)]) for n in range(N)]

            @plsc.parallel_loop(0, oc16, unroll=2)
            def outchunk(c):
                o = jnp.zeros((16,), jnp.float32)
                for s in range(_S):
                    off = s * oc + c * 16
                    s0 = sup_v[pl.ds(off, 16)]
                    s1 = sup_v[pl.ds(W + off, 16)]
                    s2 = sup_v[pl.ds(2 * W + off, 16)]
                    acc = jnp.full((16,), -jnp.inf, jnp.float32)
                    for n in range(N):
                        row = buf[n, pl.ds(off, 16)]
                        d0, d1, d2 = spl[n]
                        th = jnp.maximum(d0 * s0 + d1 * s1 + d2 * s2, 0.0)
                        acc = jnp.maximum(acc, th * row)
                    o = o + acc
                out_v[pl.ds(p * oc + c * 16, 16)] = o

        start(0, rows0_v, sem0)
        if db:
            def pair(p2, carry):
                p = p2 * 2
                start(p + 1, rows1_v, sem1)
                compute(p, rows0_v, sem0)
                start(p + 2, rows0_v, sem0)
                compute(p + 1, rows1_v, sem1)
                return carry

            lax.fori_loop(0, ppw // 2, pair, 0)
            pltpu.make_async_copy(table_h.at[idx_v[0, :]], rows0_v, sem0).wait()
        else:
            def point(p, carry):
                compute(p, rows0_v, sem0)
                start(p + 1, rows0_v, sem0)
                return carry

            lax.fori_loop(0, ppw, point, 0)
            pltpu.make_async_copy(table_h.at[idx_v[0, :]], rows0_v, sem0).wait()
        pltpu.sync_copy(out_v, out_h.at[pl.ds(base * oc, ppw * oc)])

    idxg = jnp.pad(idxg, ((0, 0), (0, 16 - N)))
    rows1_shape = (16, W) if db else (1, 16)
    f = pl.kernel(
        body, out_type=jax.ShapeDtypeStruct((P * oc,), jnp.float32), mesh=mesh,
        scratch_types=[pltpu.VMEM((ppw, 16), jnp.int32),
                       pltpu.VMEM((ppw * nd,), jnp.float32),
                       pltpu.VMEM((3 * W,), jnp.float32),
                       pltpu.VMEM((16, W), jnp.float32),
                       pltpu.VMEM(rows1_shape, jnp.float32),
                       pltpu.VMEM((ppw * oc,), jnp.float32),
                       pltpu.SemaphoreType.DMA,
                       pltpu.SemaphoreType.DMA])
    return f(table, idxg, disp_spl, sup_flat).reshape(P, oc)


def _sc_pool(table, idxg, N=4):
    P, C = idxg.shape[0], table.shape[1]
    ppw = P // _NW
    C16 = C // 16
    mesh = plsc.VectorSubcoreMesh(core_axis_name="c", subcore_axis_name="s", num_cores=2, num_subcores=16)

    def body(table_h, idx_h, out_h, idx_v, rows_v, out_v, sem):
        wid = lax.axis_index("s") * 2 + lax.axis_index("c")
        base = wid * ppw
        pltpu.sync_copy(idx_h.at[pl.ds(base, ppw)], idx_v)

        def point(p, carry):
            idx_row = idx_v[p, :]
            pltpu.async_copy(table_h.at[idx_row], rows_v, sem).wait()

            def chunk(j, c2):
                acc = rows_v[0, pl.ds(j * 16, 16)]
                for n in range(1, N):
                    acc = jnp.maximum(acc, rows_v[n, pl.ds(j * 16, 16)])
                out_v[pl.ds(p * C + j * 16, 16)] = acc
                return c2

            lax.fori_loop(0, C16, chunk, 0)
            return carry

        lax.fori_loop(0, ppw, point, 0)
        pltpu.sync_copy(out_v, out_h.at[pl.ds(base * C, ppw * C)])

    idxg = jnp.pad(idxg, ((0, 0), (0, 16 - N)))
    f = pl.kernel(
        body, out_type=jax.ShapeDtypeStruct((P * C,), jnp.float32), mesh=mesh,
        scratch_types=[pltpu.VMEM((ppw, 16), jnp.int32),
                       pltpu.VMEM((16, C), jnp.float32),
                       pltpu.VMEM((ppw * C,), jnp.float32),
                       pltpu.SemaphoreType.DMA])
    return f(table, idxg).reshape(P, C)


def _sc_gather(table, idxg):
    P, C = idxg.shape[0], table.shape[1]
    ppw = P // _NW
    mesh = plsc.VectorSubcoreMesh(core_axis_name="c", subcore_axis_name="s", num_cores=2, num_subcores=16)

    def body(table_h, idx_h, out_h, idx_v, rows_v, sem):
        wid = lax.axis_index("s") * 2 + lax.axis_index("c")
        base = wid * ppw
        pltpu.sync_copy(idx_h.at[pl.ds(base, ppw)], idx_v)
        pltpu.async_copy(table_h.at[idx_v], rows_v, sem).wait()
        pltpu.sync_copy(rows_v, out_h.at[pl.ds(base, ppw)])

    f = pl.kernel(
        body, out_type=jax.ShapeDtypeStruct((P, C), jnp.float32), mesh=mesh,
        scratch_types=[pltpu.VMEM((ppw,), jnp.int32),
                       pltpu.VMEM((ppw, C), jnp.float32),
                       pltpu.SemaphoreType.DMA])
    return f(table, idxg)


def _norm_cols(x):
    return x / jnp.maximum(jnp.linalg.norm(x, axis=0, keepdims=True), 1e-12)


def _disp_of(verts, idx):
    nb = jnp.take_along_axis(verts[:, :, None, :], idx[..., None], axis=1)
    d = nb - verts[:, :, None, :]
    return d / jnp.maximum(jnp.linalg.norm(d, axis=-1, keepdims=True), 1e-12)


def _splat(d):
    B, V, N, _ = d.shape
    return jnp.broadcast_to(d[..., None], (B, V, N, 3, 16)).reshape(B * V * N * 48)


def _goffs(idx, V):
    B = idx.shape[0]
    off = (jnp.arange(B, dtype=jnp.int32) * V)[:, None, None]
    return (idx + off).reshape(B * idx.shape[1], -1)


def _layer(verts, idx, fmap, w, b, dirs, ste_w, c2_w, c2_b, bn_g, bn_b, oc, do_bn):
    B, V, ic = fmap.shape
    P = B * V
    fc, fs, fe = _pre(fmap.reshape(P, ic), w, b, ste_w, oc)
    disp = _disp_of(verts, idx)
    act = _sc_conv(fs, _goffs(idx, V), _splat(disp),
                   _norm_cols(dirs).reshape(-1), _NEI, _S * oc, oc)
    return _post(act.reshape(B, V, oc), fc.reshape(B, V, oc),
                 fe.reshape(B, V, oc), c2_w, c2_b, bn_g, bn_b, do_bn)


def kernel(vertices, cat_id, clip_r_feat, clip_t_feat, d0, ste0_w, w1, b1, dir1,
           ste1_w, c21_w, c21_b, bn1_g, bn1_b, w2, b2, dir2, ste2_w, c22_w,
           c22_b, bn2_g, bn2_b, w3, b3, dir3, ste3_w, c23_w, c23_b, bn3_g,
           bn3_b, w4, b4, dir4, ste4_w, c24_w, c24_b):
    B, V, _ = vertices.shape
    idxA = _knn10(vertices)

    dispA = _disp_of(vertices, idxA)
    disp_pad = jnp.pad(dispA.reshape(B, V, _NEI * 3), ((0, 0), (0, 0), (0, 2)))
    fm0 = _surface(disp_pad, _norm_cols(d0), vertices, ste0_w)

    fm1 = _layer(vertices, idxA, fm0, w1, b1, dir1, ste1_w, c21_w, c21_b,
                 bn1_g, bn1_b, 128, True)

    s1 = _sample(100, V, V // 4)
    fmp1 = _sc_pool(fm1.reshape(B * V, 128), _goffs(idxA[:, s1, :4], V))
    vp1 = vertices[:, s1, :]
    V1 = V // 4

    idxB = _knn10(vp1)
    fm2 = _layer(vp1, idxB, fmp1.reshape(B, V1, 128), w2, b2, dir2, ste2_w,
                 c22_w, c22_b, bn2_g, bn2_b, 256, True)
    fm3 = _layer(vp1, idxB, fm2, w3, b3, dir3, ste3_w, c23_w, c23_b,
                 bn3_g, bn3_b, 256, True)

    s2 = _sample(101, V1, V1 // 4)
    fmp2 = _sc_pool(fm3.reshape(B * V1, 256), _goffs(idxB[:, s2, :4], V1))
    vp2 = vp1[:, s2, :]
    V2 = V1 // 4

    idxC = _knn10(vp2)
    fm4 = _layer(vp2, idxC, fmp2.reshape(B, V2, 256), w4, b4, dir4, ste4_w,
                 c24_w, c24_b, None, None, 512, False)

    np1 = _nearest(vertices, vp1)
    np2 = _nearest(vertices, vp2)
    t23 = jnp.concatenate([fm2, fm3], axis=-1).reshape(B * V1, 512)
    fm23u = _sc_gather(t23, _goffs(np1[..., None], V1)[:, 0])
    fm4u = _sc_gather(fm4.reshape(B * V2, 512), _goffs(np2[..., None], V2)[:, 0])

    oh = (cat_id == jnp.arange(_OBJ_C, dtype=cat_id.dtype)[None, :]).astype(jnp.float32)
    oh = jnp.broadcast_to(oh[:, None, :], (B, V, _OBJ_C))
    return jnp.concatenate([fm0, fm1, fm23u.reshape(B, V, 512),
                            fm4u.reshape(B, V, 512), oh], axis=2)


# speedup vs baseline: 1.5594x; 1.0135x over previous
import functools

import jax
import jax.numpy as jnp
import numpy as np
from jax import lax
from jax.experimental import pallas as pl
from jax.experimental.pallas import tpu as pltpu
from jax.experimental.pallas import tpu_sc as plsc

_NEI = 10
_S = 7
_OBJ_C = 6
_NW = 32

def _sample(seed, n, k):
    return jax.random.permutation(jax.random.key(seed), n)[:k]


def _knn_body(vq_ref, va_ref, out_ref, *, V):
    vq = vq_ref[0]
    vaT = va_ref[0]
    q2 = jnp.sum(vq * vq, axis=1)[:, None]
    s2 = jnp.sum(vaT * vaT, axis=0)[None, :]
    inner = lax.dot_general(vq.astype(jnp.bfloat16), vaT.astype(jnp.bfloat16),
                            (((1,), (0,)), ((), ())),
                            preferred_element_type=jnp.float32)
    dist = (-2.0 * inner + s2) + q2
    col = lax.broadcasted_iota(jnp.int32, (128, V), 1)
    for it in range(_NEI + 1):
        m = jnp.min(dist, axis=1, keepdims=True)
        cand = jnp.where(dist == m, col, V)
        a = jnp.min(cand, axis=1)
        if it > 0:
            out_ref[0, it - 1, :] = a
        dist = jnp.where(col == a[:, None], jnp.inf, dist)
    zero = jnp.zeros((128,), jnp.int32)
    for it in range(_NEI, 16):
        out_ref[0, it, :] = zero


def _knn10(verts):
    B, V, _ = verts.shape
    Vb = V // 128
    out = pl.pallas_call(
        functools.partial(_knn_body, V=V),
        grid=(B, Vb),
        in_specs=[pl.BlockSpec((1, 128, 3), lambda b, v: (b, v, 0)),
                  pl.BlockSpec((1, 3, V), lambda b, v: (b, 0, 0))],
        out_specs=pl.BlockSpec((1, 16, 128), lambda b, v, _Vb=Vb: (b * _Vb + v, 0, 0)),
        out_shape=jax.ShapeDtypeStruct((B * Vb, 16, 128), jnp.int32),
    )(verts, verts.transpose(0, 2, 1))
    idx = out.reshape(B, Vb, 16, 128).transpose(0, 1, 3, 2).reshape(B, V, 16)
    return idx[:, :, :_NEI]


def _nearest_body(vq_ref, vs_ref, out_ref, *, Vs):
    vq = vq_ref[0]
    vsT = vs_ref[0]
    q2 = jnp.sum(vq * vq, axis=1)[:, None]
    s2 = jnp.sum(vsT * vsT, axis=0)[None, :]
    inner = lax.dot_general(vq.astype(jnp.bfloat16), vsT.astype(jnp.bfloat16),
                            (((1,), (0,)), ((), ())),
                            preferred_element_type=jnp.float32)
    dist = (s2 + q2) - 2.0 * inner
    col = lax.broadcasted_iota(jnp.int32, (128, Vs), 1)
    m = jnp.min(dist, axis=1, keepdims=True)
    a = jnp.min(jnp.where(dist == m, col, Vs), axis=1)
    out_ref[0, 0, :] = a


def _nearest(target, source):
    B, Vt, _ = target.shape
    Vs = source.shape[1]
    Vb = Vt // 128
    out = pl.pallas_call(
        functools.partial(_nearest_body, Vs=Vs),
        grid=(B, Vb),
        in_specs=[pl.BlockSpec((1, 128, 3), lambda b, v: (b, v, 0)),
                  pl.BlockSpec((1, 3, Vs), lambda b, v: (b, 0, 0))],
        out_specs=pl.BlockSpec((1, 1, 128), lambda b, v, _Vb=Vb: (b * _Vb + v, 0, 0)),
        out_shape=jax.ShapeDtypeStruct((B * Vb, 1, 128), jnp.int32),
    )(target, source.transpose(0, 2, 1))
    return out.reshape(B, Vt)


def _surf_body(disp_ref, sup_ref, vq_ref, ste_ref, out_ref):
    disp = disp_ref[0]
    sup = sup_ref[...]
    acc = jnp.full((256, 896), -jnp.inf, jnp.float32)
    for n in range(_NEI):
        dn = disp[:, 3 * n:3 * n + 3]
        th = lax.dot_general(dn, sup, (((1,), (0,)), ((), ())),
                             preferred_element_type=jnp.float32)
        acc = jnp.maximum(acc, jnp.maximum(th, 0.0))
    feat = acc[:, 0:128]
    for s in range(1, _S):
        feat = feat + acc[:, s * 128:(s + 1) * 128]
    f_ste = lax.dot_general(vq_ref[0], ste_ref[...], (((1,), (0,)), ((), ())),
                            preferred_element_type=jnp.float32)
    out_ref[0] = jnp.maximum(feat + f_ste, 0.0)


def _surface(disp_pad, sup, verts, ste0_w):
    B, V, _ = verts.shape
    return pl.pallas_call(
        _surf_body,
        grid=(B, V // 256),
        in_specs=[pl.BlockSpec((1, 256, 32), lambda b, v: (b, v, 0)),
                  pl.BlockSpec((3, _S * 128), lambda b, v: (0, 0)),
                  pl.BlockSpec((1, 256, 3), lambda b, v: (b, v, 0)),
                  pl.BlockSpec((3, 128), lambda b, v: (0, 0))],
        out_specs=pl.BlockSpec((1, 256, 128), lambda b, v: (b, v, 0)),
        out_shape=jax.ShapeDtypeStruct((B, V, 128), jnp.float32),
    )(disp_pad, sup, verts, ste0_w.T)


def _pre_body(x_ref, w_ref, b_ref, ste_ref, fc_ref, fs_ref, fe_ref, *, oc):
    x = x_ref[...]
    fo = jnp.dot(x, w_ref[...], preferred_element_type=jnp.float32) + b_ref[...]
    fc_ref[...] = fo[:, :oc]
    fs_ref[...] = fo[:, oc:]
    fe_ref[...] = jnp.dot(x, ste_ref[...], preferred_element_type=jnp.float32)


def _pre(x, w, b, ste_w, oc):
    P, ic = x.shape
    K = w.shape[1]
    R = min(P, 512)
    return pl.pallas_call(
        functools.partial(_pre_body, oc=oc),
        grid=(P // R,),
        in_specs=[pl.BlockSpec((R, ic), lambda r: (r, 0)),
                  pl.BlockSpec((ic, K), lambda r: (0, 0)),
                  pl.BlockSpec((1, K), lambda r: (0, 0)),
                  pl.BlockSpec((ic, oc), lambda r: (0, 0))],
        out_specs=[pl.BlockSpec((R, oc), lambda r: (r, 0)),
                   pl.BlockSpec((R, K - oc), lambda r: (r, 0)),
                   pl.BlockSpec((R, oc), lambda r: (r, 0))],
        out_shape=[jax.ShapeDtypeStruct((P, oc), jnp.float32),
                   jax.ShapeDtypeStruct((P, K - oc), jnp.float32),
                   jax.ShapeDtypeStruct((P, oc), jnp.float32)],
    )(x, w, b.reshape(1, K), ste_w.T)


def _post_body(act_ref, fc_ref, fe_ref, a_ref, bm_ref, c2b_ref, g_ref, b_ref,
               out_ref, *, B, V, oc, do_bn):
    fuse = fc_ref[...] + act_ref[...]
    f2 = fuse.reshape(B * V, oc)
    gm = jnp.mean(fuse, axis=1)
    y2 = jnp.dot(f2, a_ref[...], preferred_element_type=jnp.float32)
    g2 = jnp.dot(gm, bm_ref[...], preferred_element_type=jnp.float32)
    feat = (y2.reshape(B, V, oc) + g2[:, None, :] + c2b_ref[...][None]
            + fuse + fe_ref[...])
    if do_bn:
        fr = feat.reshape(B * V, oc)
        m = jnp.mean(fr, axis=0)
        v = jnp.mean((fr - m[None, :]) ** 2, axis=0)
        feat = (feat - m[None, None, :]) / jnp.sqrt(v + 1e-5)[None, None, :]
        feat = feat * g_ref[...][None] + b_ref[...][None]
        feat = jnp.maximum(feat, 0.0)
    out_ref[...] = feat


def _post(act, fc, fe, c2_w, c2_b, bn_g, bn_b, do_bn):
    B, V, oc = act.shape
    a = c2_w[:, :oc].T
    bm = c2_w[:, oc:].T
    if bn_g is None:
        bn_g = jnp.ones((oc,), jnp.float32)
        bn_b = jnp.zeros((oc,), jnp.float32)
    return pl.pallas_call(
        functools.partial(_post_body, B=B, V=V, oc=oc, do_bn=do_bn),
        in_specs=[pl.BlockSpec((B, V, oc), lambda: (0, 0, 0)),
                  pl.BlockSpec((B, V, oc), lambda: (0, 0, 0)),
                  pl.BlockSpec((B, V, oc), lambda: (0, 0, 0)),
                  pl.BlockSpec((oc, oc), lambda: (0, 0)),
                  pl.BlockSpec((oc, oc), lambda: (0, 0)),
                  pl.BlockSpec((1, oc), lambda: (0, 0)),
                  pl.BlockSpec((1, oc), lambda: (0, 0)),
                  pl.BlockSpec((1, oc), lambda: (0, 0))],
        out_specs=pl.BlockSpec((B, V, oc), lambda: (0, 0, 0)),
        out_shape=jax.ShapeDtypeStruct((B, V, oc), jnp.float32),
    )(act, fc, fe, a, bm, c2_b.reshape(1, oc), bn_g.reshape(1, oc),
      bn_b.reshape(1, oc))


def _sc_conv(table, idxg, disp_spl, sup_flat, N, W, oc):
    P = idxg.shape[0]
    ppw = P // _NW
    oc16 = oc // 16
    nd = N * 48
    db = (2 * 16 * W + ppw * nd + 3 * W + ppw * oc + ppw * 16) * 4 < 500_000
    mesh = plsc.VectorSubcoreMesh(core_axis_name="c", subcore_axis_name="s", num_cores=2, num_subcores=16)

    def body(table_h, idx_h, disp_h, sup_h, out_h,
             idx_v, disp_v, sup_v, rows0_v, rows1_v, out_v, sem0, sem1):
        wid = lax.axis_index("s") * 2 + lax.axis_index("c")
        base = wid * ppw
        pltpu.sync_copy(idx_h.at[pl.ds(base, ppw)], idx_v)
        pltpu.sync_copy(disp_h.at[pl.ds(base * nd, ppw * nd)], disp_v)
        pltpu.sync_copy(sup_h, sup_v)

        def start(p, buf, sem):
            pc = jnp.minimum(p, ppw - 1)
            pltpu.make_async_copy(table_h.at[idx_v[pc, :]], buf, sem).start()

        def compute(p, buf, sem):
            pltpu.make_async_copy(table_h.at[idx_v[0, :]], buf, sem).wait()
            dbase = p * nd
            spl = [(disp_v[pl.ds(dbase + n * 48, 16)],
                    disp_v[pl.ds(dbase + n * 48 + 16, 16)],
                    disp_v[pl.ds(dbase + n * 48 + 32, 16)]) for n in range(N)]

            @plsc.parallel_loop(0, oc16, unroll=4)
            def outchunk(c):
                o = jnp.zeros((16,), jnp.float32)
                for s in range(_S):
                    off = s * oc + c * 16
                    s0 = sup_v[pl.ds(off, 16)]
                    s1 = sup_v[pl.ds(W + off, 16)]
                    s2 = sup_v[pl.ds(2 * W + off, 16)]
                    acc = jnp.full((16,), -jnp.inf, jnp.float32)
                    for n in range(N):
                        row = buf[n, pl.ds(off, 16)]
                        d0, d1, d2 = spl[n]
                        th = jnp.maximum(d0 * s0 + d1 * s1 + d2 * s2, 0.0)
                        acc = jnp.maximum(acc, th * row)
                    o = o + acc
                out_v[pl.ds(p * oc + c * 16, 16)] = o

        start(0, rows0_v, sem0)
        if db:
            def pair(p2, carry):
                p = p2 * 2
                start(p + 1, rows1_v, sem1)
                compute(p, rows0_v, sem0)
                start(p + 2, rows0_v, sem0)
                compute(p + 1, rows1_v, sem1)
                return carry

            lax.fori_loop(0, ppw // 2, pair, 0)
            pltpu.make_async_copy(table_h.at[idx_v[0, :]], rows0_v, sem0).wait()
        else:
            def point(p, carry):
                compute(p, rows0_v, sem0)
                start(p + 1, rows0_v, sem0)
                return carry

            lax.fori_loop(0, ppw, point, 0)
            pltpu.make_async_copy(table_h.at[idx_v[0, :]], rows0_v, sem0).wait()
        pltpu.sync_copy(out_v, out_h.at[pl.ds(base * oc, ppw * oc)])

    idxg = jnp.pad(idxg, ((0, 0), (0, 16 - N)))
    rows1_shape = (16, W) if db else (1, 16)
    f = pl.kernel(
        body, out_type=jax.ShapeDtypeStruct((P * oc,), jnp.float32), mesh=mesh,
        scratch_types=[pltpu.VMEM((ppw, 16), jnp.int32),
                       pltpu.VMEM((ppw * nd,), jnp.float32),
                       pltpu.VMEM((3 * W,), jnp.float32),
                       pltpu.VMEM((16, W), jnp.float32),
                       pltpu.VMEM(rows1_shape, jnp.float32),
                       pltpu.VMEM((ppw * oc,), jnp.float32),
                       pltpu.SemaphoreType.DMA,
                       pltpu.SemaphoreType.DMA])
    return f(table, idxg, disp_spl, sup_flat).reshape(P, oc)


def _sc_pool(table, idxg, N=4):
    P, C = idxg.shape[0], table.shape[1]
    ppw = P // _NW
    C16 = C // 16
    mesh = plsc.VectorSubcoreMesh(core_axis_name="c", subcore_axis_name="s", num_cores=2, num_subcores=16)

    def body(table_h, idx_h, out_h, idx_v, rows_v, out_v, sem):
        wid = lax.axis_index("s") * 2 + lax.axis_index("c")
        base = wid * ppw
        pltpu.sync_copy(idx_h.at[pl.ds(base, ppw)], idx_v)

        def point(p, carry):
            idx_row = idx_v[p, :]
            pltpu.async_copy(table_h.at[idx_row], rows_v, sem).wait()

            def chunk(j, c2):
                acc = rows_v[0, pl.ds(j * 16, 16)]
                for n in range(1, N):
                    acc = jnp.maximum(acc, rows_v[n, pl.ds(j * 16, 16)])
                out_v[pl.ds(p * C + j * 16, 16)] = acc
                return c2

            lax.fori_loop(0, C16, chunk, 0)
            return carry

        lax.fori_loop(0, ppw, point, 0)
        pltpu.sync_copy(out_v, out_h.at[pl.ds(base * C, ppw * C)])

    idxg = jnp.pad(idxg, ((0, 0), (0, 16 - N)))
    f = pl.kernel(
        body, out_type=jax.ShapeDtypeStruct((P * C,), jnp.float32), mesh=mesh,
        scratch_types=[pltpu.VMEM((ppw, 16), jnp.int32),
                       pltpu.VMEM((16, C), jnp.float32),
                       pltpu.VMEM((ppw * C,), jnp.float32),
                       pltpu.SemaphoreType.DMA])
    return f(table, idxg).reshape(P, C)


def _sc_gather(table, idxg):
    P, C = idxg.shape[0], table.shape[1]
    ppw = P // _NW
    mesh = plsc.VectorSubcoreMesh(core_axis_name="c", subcore_axis_name="s", num_cores=2, num_subcores=16)

    def body(table_h, idx_h, out_h, idx_v, rows_v, sem):
        wid = lax.axis_index("s") * 2 + lax.axis_index("c")
        base = wid * ppw
        pltpu.sync_copy(idx_h.at[pl.ds(base, ppw)], idx_v)
        pltpu.async_copy(table_h.at[idx_v], rows_v, sem).wait()
        pltpu.sync_copy(rows_v, out_h.at[pl.ds(base, ppw)])

    f = pl.kernel(
        body, out_type=jax.ShapeDtypeStruct((P, C), jnp.float32), mesh=mesh,
        scratch_types=[pltpu.VMEM((ppw,), jnp.int32),
                       pltpu.VMEM((ppw, C), jnp.float32),
                       pltpu.SemaphoreType.DMA])
    return f(table, idxg)


def _norm_cols(x):
    return x / jnp.maximum(jnp.linalg.norm(x, axis=0, keepdims=True), 1e-12)


def _disp_of(verts, idx):
    nb = jnp.take_along_axis(verts[:, :, None, :], idx[..., None], axis=1)
    d = nb - verts[:, :, None, :]
    return d / jnp.maximum(jnp.linalg.norm(d, axis=-1, keepdims=True), 1e-12)


def _splat(d):
    B, V, N, _ = d.shape
    return jnp.broadcast_to(d[..., None], (B, V, N, 3, 16)).reshape(B * V * N * 48)


def _goffs(idx, V):
    B = idx.shape[0]
    off = (jnp.arange(B, dtype=jnp.int32) * V)[:, None, None]
    return (idx + off).reshape(B * idx.shape[1], -1)


def _layer(verts, idx, fmap, w, b, dirs, ste_w, c2_w, c2_b, bn_g, bn_b, oc, do_bn):
    B, V, ic = fmap.shape
    P = B * V
    fc, fs, fe = _pre(fmap.reshape(P, ic), w, b, ste_w, oc)
    disp = _disp_of(verts, idx)
    act = _sc_conv(fs, _goffs(idx, V), _splat(disp),
                   _norm_cols(dirs).reshape(-1), _NEI, _S * oc, oc)
    return _post(act.reshape(B, V, oc), fc.reshape(B, V, oc),
                 fe.reshape(B, V, oc), c2_w, c2_b, bn_g, bn_b, do_bn)


def kernel(vertices, cat_id, clip_r_feat, clip_t_feat, d0, ste0_w, w1, b1, dir1,
           ste1_w, c21_w, c21_b, bn1_g, bn1_b, w2, b2, dir2, ste2_w, c22_w,
           c22_b, bn2_g, bn2_b, w3, b3, dir3, ste3_w, c23_w, c23_b, bn3_g,
           bn3_b, w4, b4, dir4, ste4_w, c24_w, c24_b):
    B, V, _ = vertices.shape
    idxA = _knn10(vertices)

    dispA = _disp_of(vertices, idxA)
    disp_pad = jnp.pad(dispA.reshape(B, V, _NEI * 3), ((0, 0), (0, 0), (0, 2)))
    fm0 = _surface(disp_pad, _norm_cols(d0), vertices, ste0_w)

    fm1 = _layer(vertices, idxA, fm0, w1, b1, dir1, ste1_w, c21_w, c21_b,
                 bn1_g, bn1_b, 128, True)

    s1 = _sample(100, V, V // 4)
    fmp1 = _sc_pool(fm1.reshape(B * V, 128), _goffs(idxA[:, s1, :4], V))
    vp1 = vertices[:, s1, :]
    V1 = V // 4

    idxB = _knn10(vp1)
    fm2 = _layer(vp1, idxB, fmp1.reshape(B, V1, 128), w2, b2, dir2, ste2_w,
                 c22_w, c22_b, bn2_g, bn2_b, 256, True)
    fm3 = _layer(vp1, idxB, fm2, w3, b3, dir3, ste3_w, c23_w, c23_b,
                 bn3_g, bn3_b, 256, True)

    s2 = _sample(101, V1, V1 // 4)
    fmp2 = _sc_pool(fm3.reshape(B * V1, 256), _goffs(idxB[:, s2, :4], V1))
    vp2 = vp1[:, s2, :]
    V2 = V1 // 4

    idxC = _knn10(vp2)
    fm4 = _layer(vp2, idxC, fmp2.reshape(B, V2, 256), w4, b4, dir4, ste4_w,
                 c24_w, c24_b, None, None, 512, False)

    np1 = _nearest(vertices, vp1)
    np2 = _nearest(vertices, vp2)
    t23 = jnp.concatenate([fm2, fm3], axis=-1).reshape(B * V1, 512)
    fm23u = _sc_gather(t23, _goffs(np1[..., None], V1)[:, 0])
    fm4u = _sc_gather(fm4.reshape(B * V2, 512), _goffs(np2[..., None], V2)[:, 0])

    oh = (cat_id == jnp.arange(_OBJ_C, dtype=cat_id.dtype)[None, :]).astype(jnp.float32)
    oh = jnp.broadcast_to(oh[:, None, :], (B, V, _OBJ_C))
    return jnp.concatenate([fm0, fm1, fm23u.reshape(B, V, 512),
                            fm4u.reshape(B, V, 512), oh], axis=2)
